# baseline (speedup 1.0000x reference)
import functools

import jax
import jax.numpy as jnp
from jax import lax
from jax.experimental import pallas as pl
from jax.experimental.pallas import tpu as pltpu
from jax.experimental.pallas import tpu_sc as plsc

B, T, D = 4, 2048, 1024
H = D // 2
NUM_SLOTS = 64
TOP_K = 8
TBLK = 512
NT = T // TBLK
NEG = -1e30
NC, NS, L = 2, 16, 16
NW = NC * NS
CHUNK = B * T // NW
NG = CHUNK // L


def _routing_kernel(x_ref, w1a_ref, c1_ref, w2_ref, b2_ref, den_ref, lg_ref):
    xb = x_ref[0]
    h1t = jax.lax.dot_general(w1a_ref[...], xb, (((1,), (1,)), ((), ())),
                              preferred_element_type=jnp.float32)
    h1t = h1t + c1_ref[...]
    h1t = 0.5 * h1t * (1.0 + jax.lax.erf(h1t * 0.7071067811865476))
    lgt = jax.lax.dot_general(w2_ref[...], h1t, (((1,), (0,)), ((), ())),
                              preferred_element_type=jnp.float32)
    lg_ref[0] = (lgt + b2_ref[...]) / den_ref[...]


def _sc_topk_kernel(lgt_hbm, alpha_hbm, lg_v, al_v):
    wid = lax.axis_index("s") * NC + lax.axis_index("c")
    b = wid // (T // CHUNK)
    t0 = (wid % (T // CHUNK)) * CHUNK
    pltpu.sync_copy(lgt_hbm.at[b, :, pl.ds(t0, CHUNK)], lg_v)

    def group(g, carry):
        base = g * L
        idxs, ws = [], []
        v0 = None
        prev = None
        denom = jnp.zeros((L,), jnp.float32)
        for k in range(TOP_K):
            m = jnp.full((L,), NEG, jnp.float32)
            idxf = jnp.zeros((L,), jnp.float32)
            for s in range(NUM_SLOTS):
                v = lg_v[s, pl.ds(base, L)]
                if k > 0:
                    v = jnp.where(prev == float(s), NEG, v)
                    lg_v[s, pl.ds(base, L)] = v
                gt = v > m
                m = jnp.where(gt, v, m)
                idxf = jnp.where(gt, float(s), idxf)
            prev = idxf
            if k == 0:
                v0 = m
            w = jnp.exp(m - v0)
            denom = denom + w
            idxs.append(idxf)
            ws.append(w)
        wn = [w / denom for w in ws]
        for s in range(NUM_SLOTS):
            a = jnp.zeros((L,), jnp.float32)
            for k in range(TOP_K):
                a = jnp.where(idxs[k] == float(s), wn[k], a)
            al_v[s, pl.ds(base, L)] = a
        return carry

    lax.fori_loop(0, NG, group, 0)
    pltpu.sync_copy(al_v, alpha_hbm.at[b, :, pl.ds(t0, CHUNK)])


def _dispatch_kernel(alpha_ref, x_ref, si_ref):
    t = pl.program_id(1)
    contrib = jax.lax.dot_general(alpha_ref[0], x_ref[0],
                                  (((1,), (0,)), ((), ())),
                                  preferred_element_type=jnp.float32)

    @pl.when(t == 0)
    def _():
        si_ref[0] = contrib

    @pl.when(t != 0)
    def _():
        si_ref[0] += contrib


def _nt(a, b):
    return jax.lax.dot_general(a, b, (((1,), (1,)), ((), ())),
                               preferred_element_type=jnp.float32)


def _gru_kernel(si_ref, alpha_ref, hp0_ref, wih_ref, whh_ref, bih_ref,
                bhh_ref, wv_ref, wo_ref, bv_ref, svo_ref):
    ws = jnp.sum(alpha_ref[...], axis=2) + 1e-8
    xi = (si_ref[...] / ws[..., None]).reshape(B * NUM_SLOTS, D)
    gi = _nt(xi, wih_ref[...]) + bih_ref[...]
    hp0 = hp0_ref[...]
    gh0 = _nt(hp0, whh_ref[...]) + bhh_ref[...]
    gh = jnp.broadcast_to(gh0[None], (B, NUM_SLOTS, 3 * D))
    gh = gh.reshape(B * NUM_SLOTS, 3 * D)
    hp = jnp.broadcast_to(hp0[None], (B, NUM_SLOTS, D)).reshape(B * NUM_SLOTS, D)
    r = jax.nn.sigmoid(gi[:, :D] + gh[:, :D])
    z = jax.nn.sigmoid(gi[:, D:2 * D] + gh[:, D:2 * D])
    n = jnp.tanh(gi[:, 2 * D:] + r * gh[:, 2 * D:])
    hnew = (1.0 - z) * n + z * hp
    sv = _nt(hnew, wv_ref[...]) + bv_ref[...]
    svo = _nt(sv, wo_ref[...]).reshape(B, NUM_SLOTS, D)
    svo_ref[...] = svo


def _combine_kernel(alpha_ref, svo_ref, bo_ref, out_ref):
    out = jax.lax.dot_general(alpha_ref[0], svo_ref[0],
                              (((0,), (0,)), ((), ())),
                              preferred_element_type=jnp.float32)
    out_ref[0] = out + bo_ref[...]


@jax.jit
def kernel(x, slot_init, slot_scale, W1, b1, W2, b2, W_ih, W_hh, b_ih, b_hh,
           Wv, bv, Wo, bo, tau):
    f32 = jnp.float32
    hp0 = slot_init * slot_scale
    slot_mean = jnp.mean(hp0, axis=0)
    c1 = (b1 + W1[:, D:] @ slot_mean).reshape(H, 1)
    den = (jnp.abs(tau) + 0.1).reshape(1, 1)

    lgt = pl.pallas_call(
        _routing_kernel,
        grid=(B, NT),
        in_specs=[
            pl.BlockSpec((1, TBLK, D), lambda b, t: (b, t, 0)),
            pl.BlockSpec((H, D), lambda b, t: (0, 0)),
            pl.BlockSpec((H, 1), lambda b, t: (0, 0)),
            pl.BlockSpec((NUM_SLOTS, H), lambda b, t: (0, 0)),
            pl.BlockSpec((NUM_SLOTS, 1), lambda b, t: (0, 0)),
            pl.BlockSpec((1, 1), lambda b, t: (0, 0)),
        ],
        out_specs=pl.BlockSpec((1, NUM_SLOTS, TBLK), lambda b, t: (b, 0, t)),
        out_shape=jax.ShapeDtypeStruct((B, NUM_SLOTS, T), f32),
    )(x, W1[:, :D], c1, W2, b2.reshape(NUM_SLOTS, 1), den)

    sc_topk = functools.partial(
        pl.kernel,
        mesh=plsc.VectorSubcoreMesh(core_axis_name="c", subcore_axis_name="s"),
        out_type=jax.ShapeDtypeStruct((B, NUM_SLOTS, T), f32),
        scratch_types=[
            pltpu.VMEM((NUM_SLOTS, CHUNK), f32),
            pltpu.VMEM((NUM_SLOTS, CHUNK), f32),
        ],
    )(_sc_topk_kernel)
    alpha_t = sc_topk(lgt)

    slot_input = pl.pallas_call(
        _dispatch_kernel,
        grid=(B, NT),
        in_specs=[
            pl.BlockSpec((1, NUM_SLOTS, TBLK), lambda b, t: (b, 0, t)),
            pl.BlockSpec((1, TBLK, D), lambda b, t: (b, t, 0)),
        ],
        out_specs=pl.BlockSpec((1, NUM_SLOTS, D), lambda b, t: (b, 0, 0)),
        out_shape=jax.ShapeDtypeStruct((B, NUM_SLOTS, D), f32),
    )(alpha_t, x)

    svo = pl.pallas_call(
        _gru_kernel,
        out_shape=jax.ShapeDtypeStruct((B, NUM_SLOTS, D), f32),
    )(slot_input, alpha_t, hp0, W_ih, W_hh, b_ih.reshape(1, 3 * D),
      b_hh.reshape(1, 3 * D), Wv, Wo, bv.reshape(1, D))

    out = pl.pallas_call(
        _combine_kernel,
        grid=(B, NT),
        in_specs=[
            pl.BlockSpec((1, NUM_SLOTS, TBLK), lambda b, t: (b, 0, t)),
            pl.BlockSpec((1, NUM_SLOTS, D), lambda b, t: (b, 0, 0)),
            pl.BlockSpec((1, D), lambda b, t: (0, 0)),
        ],
        out_specs=pl.BlockSpec((1, TBLK, D), lambda b, t: (b, t, 0)),
        out_shape=jax.ShapeDtypeStruct((B, T, D), f32),
    )(alpha_t, svo, bo.reshape(1, D))
    return out


# speedup vs baseline: 1.0332x; 1.0332x over previous
import functools

import jax
import jax.numpy as jnp
from jax import lax
from jax.experimental import pallas as pl
from jax.experimental.pallas import tpu as pltpu
from jax.experimental.pallas import tpu_sc as plsc

B, T, D = 4, 2048, 1024
H = D // 2
NUM_SLOTS = 64
TOP_K = 8
TBLK = 512
NT = T // TBLK
NEG = -1e30
NC, NS, L = 2, 16, 16
NW = NC * NS
CHUNK = B * T // NW
NG = CHUNK // L


def _routing_kernel(x_ref, w1a_ref, c1_ref, w2_ref, b2_ref, den_ref, lg_ref):
    xb = x_ref[0]
    h1t = jax.lax.dot_general(w1a_ref[...], xb, (((1,), (1,)), ((), ())),
                              preferred_element_type=jnp.float32)
    h1t = h1t + c1_ref[...]
    h1t = 0.5 * h1t * (1.0 + jax.lax.erf(h1t * 0.7071067811865476))
    lgt = jax.lax.dot_general(w2_ref[...], h1t, (((1,), (0,)), ((), ())),
                              preferred_element_type=jnp.float32)
    lg_ref[0] = (lgt + b2_ref[...]) / den_ref[...]


def _sc_topk_kernel(lgt_hbm, alpha_hbm, lg_v, al_v):
    wid = lax.axis_index("s") * NC + lax.axis_index("c")
    b = wid // (T // CHUNK)
    t0 = (wid % (T // CHUNK)) * CHUNK
    pltpu.sync_copy(lgt_hbm.at[b, :, pl.ds(t0, CHUNK)], lg_v)

    def group(g, carry):
        base = g * L
        idxs, ws = [], []
        v0 = None
        prev = None
        denom = jnp.zeros((L,), jnp.float32)
        for k in range(TOP_K):
            nodes = []
            for s in range(NUM_SLOTS):
                v = lg_v[s, pl.ds(base, L)]
                if k > 0:
                    v = jnp.where(prev == float(s), NEG, v)
                    lg_v[s, pl.ds(base, L)] = v
                nodes.append((v, jnp.full((L,), float(s), jnp.float32)))
            while len(nodes) > 1:
                nxt = []
                for i in range(0, len(nodes), 2):
                    va, ia = nodes[i]
                    vb, ib = nodes[i + 1]
                    gt = vb > va
                    nxt.append((jnp.where(gt, vb, va), jnp.where(gt, ib, ia)))
                nodes = nxt
            m, idxf = nodes[0]
            prev = idxf
            if k == 0:
                v0 = m
            w = jnp.exp(m - v0)
            denom = denom + w
            idxs.append(idxf)
            ws.append(w)
        wn = [w / denom for w in ws]
        for s in range(NUM_SLOTS):
            a = jnp.zeros((L,), jnp.float32)
            for k in range(TOP_K):
                a = jnp.where(idxs[k] == float(s), wn[k], a)
            al_v[s, pl.ds(base, L)] = a
        return carry

    lax.fori_loop(0, NG, group, 0)
    pltpu.sync_copy(al_v, alpha_hbm.at[b, :, pl.ds(t0, CHUNK)])


def _dispatch_kernel(alpha_ref, x_ref, si_ref):
    t = pl.program_id(1)
    contrib = jax.lax.dot_general(alpha_ref[0], x_ref[0],
                                  (((1,), (0,)), ((), ())),
                                  preferred_element_type=jnp.float32)

    @pl.when(t == 0)
    def _():
        si_ref[0] = contrib

    @pl.when(t != 0)
    def _():
        si_ref[0] += contrib


def _nt(a, b):
    return jax.lax.dot_general(a, b, (((1,), (1,)), ((), ())),
                               preferred_element_type=jnp.float32)


def _gru_kernel(si_ref, alpha_ref, hp0_ref, wih_ref, whh_ref, bih_ref,
                bhh_ref, wv_ref, wo_ref, bv_ref, svo_ref):
    ws = jnp.sum(alpha_ref[...], axis=2) + 1e-8
    xi = (si_ref[...] / ws[..., None]).reshape(B * NUM_SLOTS, D)
    gi = _nt(xi, wih_ref[...]) + bih_ref[...]
    hp0 = hp0_ref[...]
    gh0 = _nt(hp0, whh_ref[...]) + bhh_ref[...]
    gh = jnp.broadcast_to(gh0[None], (B, NUM_SLOTS, 3 * D))
    gh = gh.reshape(B * NUM_SLOTS, 3 * D)
    hp = jnp.broadcast_to(hp0[None], (B, NUM_SLOTS, D)).reshape(B * NUM_SLOTS, D)
    r = jax.nn.sigmoid(gi[:, :D] + gh[:, :D])
    z = jax.nn.sigmoid(gi[:, D:2 * D] + gh[:, D:2 * D])
    n = jnp.tanh(gi[:, 2 * D:] + r * gh[:, 2 * D:])
    hnew = (1.0 - z) * n + z * hp
    sv = _nt(hnew, wv_ref[...]) + bv_ref[...]
    svo = _nt(sv, wo_ref[...]).reshape(B, NUM_SLOTS, D)
    svo_ref[...] = svo


def _combine_kernel(alpha_ref, svo_ref, bo_ref, out_ref):
    out = jax.lax.dot_general(alpha_ref[0], svo_ref[0],
                              (((0,), (0,)), ((), ())),
                              preferred_element_type=jnp.float32)
    out_ref[0] = out + bo_ref[...]


@jax.jit
def kernel(x, slot_init, slot_scale, W1, b1, W2, b2, W_ih, W_hh, b_ih, b_hh,
           Wv, bv, Wo, bo, tau):
    f32 = jnp.float32
    hp0 = slot_init * slot_scale
    slot_mean = jnp.mean(hp0, axis=0)
    c1 = (b1 + W1[:, D:] @ slot_mean).reshape(H, 1)
    den = (jnp.abs(tau) + 0.1).reshape(1, 1)

    lgt = pl.pallas_call(
        _routing_kernel,
        grid=(B, NT),
        in_specs=[
            pl.BlockSpec((1, TBLK, D), lambda b, t: (b, t, 0)),
            pl.BlockSpec((H, D), lambda b, t: (0, 0)),
            pl.BlockSpec((H, 1), lambda b, t: (0, 0)),
            pl.BlockSpec((NUM_SLOTS, H), lambda b, t: (0, 0)),
            pl.BlockSpec((NUM_SLOTS, 1), lambda b, t: (0, 0)),
            pl.BlockSpec((1, 1), lambda b, t: (0, 0)),
        ],
        out_specs=pl.BlockSpec((1, NUM_SLOTS, TBLK), lambda b, t: (b, 0, t)),
        out_shape=jax.ShapeDtypeStruct((B, NUM_SLOTS, T), f32),
    )(x, W1[:, :D], c1, W2, b2.reshape(NUM_SLOTS, 1), den)

    sc_topk = functools.partial(
        pl.kernel,
        mesh=plsc.VectorSubcoreMesh(core_axis_name="c", subcore_axis_name="s"),
        out_type=jax.ShapeDtypeStruct((B, NUM_SLOTS, T), f32),
        scratch_types=[
            pltpu.VMEM((NUM_SLOTS, CHUNK), f32),
            pltpu.VMEM((NUM_SLOTS, CHUNK), f32),
        ],
    )(_sc_topk_kernel)
    alpha_t = sc_topk(lgt)

    slot_input = pl.pallas_call(
        _dispatch_kernel,
        grid=(B, NT),
        in_specs=[
            pl.BlockSpec((1, NUM_SLOTS, TBLK), lambda b, t: (b, 0, t)),
            pl.BlockSpec((1, TBLK, D), lambda b, t: (b, t, 0)),
        ],
        out_specs=pl.BlockSpec((1, NUM_SLOTS, D), lambda b, t: (b, 0, 0)),
        out_shape=jax.ShapeDtypeStruct((B, NUM_SLOTS, D), f32),
    )(alpha_t, x)

    svo = pl.pallas_call(
        _gru_kernel,
        out_shape=jax.ShapeDtypeStruct((B, NUM_SLOTS, D), f32),
    )(slot_input, alpha_t, hp0, W_ih, W_hh, b_ih.reshape(1, 3 * D),
      b_hh.reshape(1, 3 * D), Wv, Wo, bv.reshape(1, D))

    out = pl.pallas_call(
        _combine_kernel,
        grid=(B, NT),
        in_specs=[
            pl.BlockSpec((1, NUM_SLOTS, TBLK), lambda b, t: (b, 0, t)),
            pl.BlockSpec((1, NUM_SLOTS, D), lambda b, t: (b, 0, 0)),
            pl.BlockSpec((1, D), lambda b, t: (0, 0)),
        ],
        out_specs=pl.BlockSpec((1, TBLK, D), lambda b, t: (b, t, 0)),
        out_shape=jax.ShapeDtypeStruct((B, T, D), f32),
    )(alpha_t, svo, bo.reshape(1, D))
    return out


# speedup vs baseline: 2.7713x; 2.6822x over previous
import functools

import jax
import jax.numpy as jnp
from jax.experimental import pallas as pl

B, T, D = 4, 2048, 1024
H = D // 2
NUM_SLOTS = 64
TOP_K = 8
TBLK = 1024
NT = T // TBLK
NEG = -1e30


def _routing_dispatch_kernel(x_ref, w1a_ref, c1_ref, w2_ref, b2_ref,
                             den_ref, alpha_ref, si_ref):
    t = pl.program_id(1)
    xb = x_ref[0]
    h1t = jax.lax.dot_general(w1a_ref[...], xb, (((1,), (1,)), ((), ())),
                              preferred_element_type=jnp.float32)
    h1t = h1t + c1_ref[...]
    h1t = 0.5 * h1t * (1.0 + jax.lax.erf(h1t * 0.7071067811865476))
    lgt = jax.lax.dot_general(w2_ref[...], h1t, (((1,), (0,)), ((), ())),
                              preferred_element_type=jnp.float32)
    lgt = (lgt + b2_ref[...]) / den_ref[...]

    iota = jax.lax.broadcasted_iota(
        jnp.int32, (NUM_SLOTS, TBLK), 0).astype(jnp.float32)
    run = lgt
    v0 = jnp.max(run, axis=0, keepdims=True)
    alpha = jnp.zeros_like(lgt)
    denom = jnp.zeros((1, TBLK), jnp.float32)
    for _ in range(TOP_K):
        v = jnp.max(run, axis=0, keepdims=True)
        eq = run == v
        idx = jnp.min(jnp.where(eq, iota, 64.0), axis=0, keepdims=True)
        onehot = iota == idx
        w = jnp.exp(v - v0)
        alpha = jnp.where(onehot, jnp.broadcast_to(w, alpha.shape), alpha)
        denom = denom + w
        run = jnp.where(onehot, NEG, run)
    alpha = alpha / denom
    alpha_ref[0] = alpha.astype(jnp.bfloat16)

    contrib = jax.lax.dot_general(alpha, xb, (((1,), (0,)), ((), ())),
                                  preferred_element_type=jnp.float32)

    @pl.when(t == 0)
    def _():
        si_ref[0] = contrib

    @pl.when(t != 0)
    def _():
        si_ref[0] += contrib


def _nt(a, b):
    return jax.lax.dot_general(a, b, (((1,), (1,)), ((), ())),
                               preferred_element_type=jnp.float32)


def _gru_kernel(si_ref, alpha_ref, hp0_ref, wih_ref, whh_ref, bih_ref,
                bhh_ref, wv_ref, wo_ref, bv_ref, svo_ref):
    ws = jnp.sum(alpha_ref[...].astype(jnp.float32), axis=2) + 1e-8
    xi = (si_ref[...] / ws[..., None]).reshape(B * NUM_SLOTS, D)
    gi = _nt(xi, wih_ref[...]) + bih_ref[...]
    hp0 = hp0_ref[...]
    gh0 = _nt(hp0, whh_ref[...]) + bhh_ref[...]
    gh = jnp.broadcast_to(gh0[None], (B, NUM_SLOTS, 3 * D))
    gh = gh.reshape(B * NUM_SLOTS, 3 * D)
    hp = jnp.broadcast_to(hp0[None], (B, NUM_SLOTS, D)).reshape(B * NUM_SLOTS, D)
    r = jax.nn.sigmoid(gi[:, :D] + gh[:, :D])
    z = jax.nn.sigmoid(gi[:, D:2 * D] + gh[:, D:2 * D])
    n = jnp.tanh(gi[:, 2 * D:] + r * gh[:, 2 * D:])
    hnew = (1.0 - z) * n + z * hp
    sv = _nt(hnew, wv_ref[...]) + bv_ref[...]
    svo = _nt(sv, wo_ref[...]).reshape(B, NUM_SLOTS, D)
    svo_ref[...] = svo.astype(jnp.bfloat16)


def _combine_kernel(alpha_ref, svo_ref, bo_ref, out_ref):
    out = jax.lax.dot_general(alpha_ref[0], svo_ref[0],
                              (((0,), (0,)), ((), ())),
                              preferred_element_type=jnp.float32)
    out_ref[0] = out + bo_ref[...]


@jax.jit
def kernel(x, slot_init, slot_scale, W1, b1, W2, b2, W_ih, W_hh, b_ih, b_hh,
           Wv, bv, Wo, bo, tau):
    f32 = jnp.float32
    bf16 = jnp.bfloat16
    hp0 = slot_init * slot_scale
    slot_mean = jnp.mean(hp0, axis=0)
    c1 = (b1 + W1[:, D:] @ slot_mean).reshape(H, 1)
    den = (jnp.abs(tau) + 0.1).reshape(1, 1)

    alpha_t, slot_input = pl.pallas_call(
        _routing_dispatch_kernel,
        grid=(B, NT),
        in_specs=[
            pl.BlockSpec((1, TBLK, D), lambda b, t: (b, t, 0)),
            pl.BlockSpec((H, D), lambda b, t: (0, 0)),
            pl.BlockSpec((H, 1), lambda b, t: (0, 0)),
            pl.BlockSpec((NUM_SLOTS, H), lambda b, t: (0, 0)),
            pl.BlockSpec((NUM_SLOTS, 1), lambda b, t: (0, 0)),
            pl.BlockSpec((1, 1), lambda b, t: (0, 0)),
        ],
        out_specs=[
            pl.BlockSpec((1, NUM_SLOTS, TBLK), lambda b, t: (b, 0, t)),
            pl.BlockSpec((1, NUM_SLOTS, D), lambda b, t: (b, 0, 0)),
        ],
        out_shape=[
            jax.ShapeDtypeStruct((B, NUM_SLOTS, T), bf16),
            jax.ShapeDtypeStruct((B, NUM_SLOTS, D), f32),
        ],
    )(x, W1[:, :D], c1, W2, b2.reshape(NUM_SLOTS, 1), den)

    svo = pl.pallas_call(
        _gru_kernel,
        out_shape=jax.ShapeDtypeStruct((B, NUM_SLOTS, D), bf16),
    )(slot_input, alpha_t, hp0, W_ih, W_hh, b_ih.reshape(1, 3 * D),
      b_hh.reshape(1, 3 * D), Wv, Wo, bv.reshape(1, D))

    out = pl.pallas_call(
        _combine_kernel,
        grid=(B, NT),
        in_specs=[
            pl.BlockSpec((1, NUM_SLOTS, TBLK), lambda b, t: (b, 0, t)),
            pl.BlockSpec((1, NUM_SLOTS, D), lambda b, t: (b, 0, 0)),
            pl.BlockSpec((1, D), lambda b, t: (0, 0)),
        ],
        out_specs=pl.BlockSpec((1, TBLK, D), lambda b, t: (b, t, 0)),
        out_shape=jax.ShapeDtypeStruct((B, T, D), f32),
    )(alpha_t, svo, bo.reshape(1, D))
    return out


# speedup vs baseline: 2.8610x; 1.0324x over previous
import functools

import jax
import jax.numpy as jnp
from jax.experimental import pallas as pl

B, T, D = 4, 2048, 1024
H = D // 2
NUM_SLOTS = 64
TOP_K = 8
TBLK = 2048
NT = T // TBLK
NEG = -1e30


def _routing_dispatch_kernel(x_ref, w1a_ref, c1_ref, w2_ref, b2_ref,
                             den_ref, alpha_ref, si_ref):
    t = pl.program_id(1)
    xb = x_ref[0]
    h1t = jax.lax.dot_general(w1a_ref[...], xb, (((1,), (1,)), ((), ())),
                              preferred_element_type=jnp.float32)
    h1t = h1t + c1_ref[...]
    h1t = 0.5 * h1t * (1.0 + jax.lax.erf(h1t * 0.7071067811865476))
    lgt = jax.lax.dot_general(w2_ref[...], h1t, (((1,), (0,)), ((), ())),
                              preferred_element_type=jnp.float32)
    lgt = (lgt + b2_ref[...]) / den_ref[...]

    iota = jax.lax.broadcasted_iota(
        jnp.int32, (NUM_SLOTS, TBLK), 0).astype(jnp.float32)
    run = lgt
    v0 = jnp.max(run, axis=0, keepdims=True)
    alpha = jnp.zeros_like(lgt)
    denom = jnp.zeros((1, TBLK), jnp.float32)
    for _ in range(TOP_K):
        v = jnp.max(run, axis=0, keepdims=True)
        eq = run == v
        idx = jnp.min(jnp.where(eq, iota, 64.0), axis=0, keepdims=True)
        onehot = iota == idx
        w = jnp.exp(v - v0)
        alpha = jnp.where(onehot, jnp.broadcast_to(w, alpha.shape), alpha)
        denom = denom + w
        run = jnp.where(onehot, NEG, run)
    alpha = alpha / denom
    alpha_ref[0] = alpha.astype(jnp.bfloat16)

    contrib = jax.lax.dot_general(alpha, xb, (((1,), (0,)), ((), ())),
                                  preferred_element_type=jnp.float32)

    @pl.when(t == 0)
    def _():
        si_ref[0] = contrib

    @pl.when(t != 0)
    def _():
        si_ref[0] += contrib


def _nt(a, b):
    return jax.lax.dot_general(a, b, (((1,), (1,)), ((), ())),
                               preferred_element_type=jnp.float32)


def _gru_kernel(si_ref, alpha_ref, hp0_ref, wih_ref, whh_ref, bih_ref,
                bhh_ref, wv_ref, wo_ref, bv_ref, svo_ref):
    ws = jnp.sum(alpha_ref[...].astype(jnp.float32), axis=2) + 1e-8
    xi = (si_ref[...] / ws[..., None]).reshape(B * NUM_SLOTS, D)
    gi = _nt(xi, wih_ref[...]) + bih_ref[...]
    hp0 = hp0_ref[...]
    gh0 = _nt(hp0, whh_ref[...]) + bhh_ref[...]
    gh = jnp.broadcast_to(gh0[None], (B, NUM_SLOTS, 3 * D))
    gh = gh.reshape(B * NUM_SLOTS, 3 * D)
    hp = jnp.broadcast_to(hp0[None], (B, NUM_SLOTS, D)).reshape(B * NUM_SLOTS, D)
    r = jax.nn.sigmoid(gi[:, :D] + gh[:, :D])
    z = jax.nn.sigmoid(gi[:, D:2 * D] + gh[:, D:2 * D])
    n = jnp.tanh(gi[:, 2 * D:] + r * gh[:, 2 * D:])
    hnew = (1.0 - z) * n + z * hp
    sv = _nt(hnew, wv_ref[...]) + bv_ref[...]
    svo = _nt(sv, wo_ref[...]).reshape(B, NUM_SLOTS, D)
    svo_ref[...] = svo.astype(jnp.bfloat16)


def _combine_kernel(alpha_ref, svo_ref, bo_ref, out_ref):
    out = jax.lax.dot_general(alpha_ref[0], svo_ref[0],
                              (((0,), (0,)), ((), ())),
                              preferred_element_type=jnp.float32)
    out_ref[0] = out + bo_ref[...]


@jax.jit
def kernel(x, slot_init, slot_scale, W1, b1, W2, b2, W_ih, W_hh, b_ih, b_hh,
           Wv, bv, Wo, bo, tau):
    f32 = jnp.float32
    bf16 = jnp.bfloat16
    hp0 = slot_init * slot_scale
    slot_mean = jnp.mean(hp0, axis=0)
    c1 = (b1 + W1[:, D:] @ slot_mean).reshape(H, 1)
    den = (jnp.abs(tau) + 0.1).reshape(1, 1)

    alpha_t, slot_input = pl.pallas_call(
        _routing_dispatch_kernel,
        grid=(B, NT),
        in_specs=[
            pl.BlockSpec((1, TBLK, D), lambda b, t: (b, t, 0)),
            pl.BlockSpec((H, D), lambda b, t: (0, 0)),
            pl.BlockSpec((H, 1), lambda b, t: (0, 0)),
            pl.BlockSpec((NUM_SLOTS, H), lambda b, t: (0, 0)),
            pl.BlockSpec((NUM_SLOTS, 1), lambda b, t: (0, 0)),
            pl.BlockSpec((1, 1), lambda b, t: (0, 0)),
        ],
        out_specs=[
            pl.BlockSpec((1, NUM_SLOTS, TBLK), lambda b, t: (b, 0, t)),
            pl.BlockSpec((1, NUM_SLOTS, D), lambda b, t: (b, 0, 0)),
        ],
        out_shape=[
            jax.ShapeDtypeStruct((B, NUM_SLOTS, T), bf16),
            jax.ShapeDtypeStruct((B, NUM_SLOTS, D), f32),
        ],
    )(x, W1[:, :D], c1, W2, b2.reshape(NUM_SLOTS, 1), den)

    svo = pl.pallas_call(
        _gru_kernel,
        out_shape=jax.ShapeDtypeStruct((B, NUM_SLOTS, D), bf16),
    )(slot_input, alpha_t, hp0, W_ih, W_hh, b_ih.reshape(1, 3 * D),
      b_hh.reshape(1, 3 * D), Wv, Wo, bv.reshape(1, D))

    out = pl.pallas_call(
        _combine_kernel,
        grid=(B, NT),
        in_specs=[
            pl.BlockSpec((1, NUM_SLOTS, TBLK), lambda b, t: (b, 0, t)),
            pl.BlockSpec((1, NUM_SLOTS, D), lambda b, t: (b, 0, 0)),
            pl.BlockSpec((1, D), lambda b, t: (0, 0)),
        ],
        out_specs=pl.BlockSpec((1, TBLK, D), lambda b, t: (b, t, 0)),
        out_shape=jax.ShapeDtypeStruct((B, T, D), f32),
    )(alpha_t, svo, bo.reshape(1, D))
    return out


# speedup vs baseline: 2.9453x; 1.0295x over previous
import functools

import jax
import jax.numpy as jnp
from jax.experimental import pallas as pl
from jax.experimental.pallas import tpu as pltpu

B, T, D = 4, 2048, 1024
H = D // 2
NUM_SLOTS = 64
TOP_K = 8
TBLK = 2048
NT = T // TBLK
NEG = -1e30


def _routing_dispatch_kernel(x_ref, w1a_ref, c1_ref, w2_ref, b2_ref,
                             den_ref, alpha_ref, si_ref):
    t = pl.program_id(1)
    xb = x_ref[0]
    h1t = jax.lax.dot_general(w1a_ref[...], xb, (((1,), (1,)), ((), ())),
                              preferred_element_type=jnp.float32)
    h1t = h1t + c1_ref[...]
    h1t = 0.5 * h1t * (1.0 + jax.lax.erf(h1t * 0.7071067811865476))
    lgt = jax.lax.dot_general(w2_ref[...], h1t, (((1,), (0,)), ((), ())),
                              preferred_element_type=jnp.float32)
    lgt = (lgt + b2_ref[...]) / den_ref[...]

    iota = jax.lax.broadcasted_iota(
        jnp.int32, (NUM_SLOTS, TBLK), 0).astype(jnp.float32)
    run = lgt
    v0 = jnp.max(run, axis=0, keepdims=True)
    alpha = jnp.zeros_like(lgt)
    denom = jnp.zeros((1, TBLK), jnp.float32)
    for _ in range(TOP_K):
        v = jnp.max(run, axis=0, keepdims=True)
        eq = run == v
        idx = jnp.min(jnp.where(eq, iota, 64.0), axis=0, keepdims=True)
        onehot = iota == idx
        w = jnp.exp(v - v0)
        alpha = jnp.where(onehot, jnp.broadcast_to(w, alpha.shape), alpha)
        denom = denom + w
        run = jnp.where(onehot, NEG, run)
    alpha = alpha / denom
    alpha_ref[0] = alpha.astype(jnp.bfloat16)

    contrib = jax.lax.dot_general(alpha, xb, (((1,), (0,)), ((), ())),
                                  preferred_element_type=jnp.float32)

    @pl.when(t == 0)
    def _():
        si_ref[0] = contrib

    @pl.when(t != 0)
    def _():
        si_ref[0] += contrib


def _nt(a, b):
    return jax.lax.dot_general(a, b, (((1,), (1,)), ((), ())),
                               preferred_element_type=jnp.float32)


def _gru_combine_kernel(alpha_blk_ref, alpha_full_ref, si_ref, hp0_ref,
                        wih_ref, whh_ref, bih_ref, bhh_ref, wv_ref, wo_ref,
                        bv_ref, bo_ref, out_ref, svo_s):
    b = pl.program_id(0)
    t = pl.program_id(1)

    @pl.when((b == 0) & (t == 0))
    def _():
        ws = jnp.sum(alpha_full_ref[...].astype(jnp.float32), axis=2) + 1e-8
        xi = (si_ref[...] / ws[..., None]).reshape(B * NUM_SLOTS, D)
        gi = _nt(xi, wih_ref[...]) + bih_ref[...]
        hp0 = hp0_ref[...]
        gh0 = _nt(hp0, whh_ref[...]) + bhh_ref[...]
        gh = jnp.broadcast_to(gh0[None], (B, NUM_SLOTS, 3 * D))
        gh = gh.reshape(B * NUM_SLOTS, 3 * D)
        hp = jnp.broadcast_to(hp0[None], (B, NUM_SLOTS, D))
        hp = hp.reshape(B * NUM_SLOTS, D)
        r = jax.nn.sigmoid(gi[:, :D] + gh[:, :D])
        z = jax.nn.sigmoid(gi[:, D:2 * D] + gh[:, D:2 * D])
        n = jnp.tanh(gi[:, 2 * D:] + r * gh[:, 2 * D:])
        hnew = (1.0 - z) * n + z * hp
        sv = _nt(hnew, wv_ref[...]) + bv_ref[...]
        svo = _nt(sv, wo_ref[...]).reshape(B, NUM_SLOTS, D)
        svo_s[...] = svo.astype(jnp.bfloat16)

    out = jax.lax.dot_general(alpha_blk_ref[0], svo_s[b],
                              (((0,), (0,)), ((), ())),
                              preferred_element_type=jnp.float32)
    out_ref[0] = out + bo_ref[...]


@jax.jit
def kernel(x, slot_init, slot_scale, W1, b1, W2, b2, W_ih, W_hh, b_ih, b_hh,
           Wv, bv, Wo, bo, tau):
    f32 = jnp.float32
    bf16 = jnp.bfloat16
    hp0 = slot_init * slot_scale
    slot_mean = jnp.mean(hp0, axis=0)
    c1 = (b1 + W1[:, D:] @ slot_mean).reshape(H, 1)
    den = (jnp.abs(tau) + 0.1).reshape(1, 1)

    alpha_t, slot_input = pl.pallas_call(
        _routing_dispatch_kernel,
        grid=(B, NT),
        in_specs=[
            pl.BlockSpec((1, TBLK, D), lambda b, t: (b, t, 0)),
            pl.BlockSpec((H, D), lambda b, t: (0, 0)),
            pl.BlockSpec((H, 1), lambda b, t: (0, 0)),
            pl.BlockSpec((NUM_SLOTS, H), lambda b, t: (0, 0)),
            pl.BlockSpec((NUM_SLOTS, 1), lambda b, t: (0, 0)),
            pl.BlockSpec((1, 1), lambda b, t: (0, 0)),
        ],
        out_specs=[
            pl.BlockSpec((1, NUM_SLOTS, TBLK), lambda b, t: (b, 0, t)),
            pl.BlockSpec((1, NUM_SLOTS, D), lambda b, t: (b, 0, 0)),
        ],
        out_shape=[
            jax.ShapeDtypeStruct((B, NUM_SLOTS, T), bf16),
            jax.ShapeDtypeStruct((B, NUM_SLOTS, D), f32),
        ],
    )(x, W1[:, :D], c1, W2, b2.reshape(NUM_SLOTS, 1), den)

    out = pl.pallas_call(
        _gru_combine_kernel,
        grid=(B, NT),
        in_specs=[
            pl.BlockSpec((1, NUM_SLOTS, TBLK), lambda b, t: (b, 0, t)),
            pl.BlockSpec((B, NUM_SLOTS, T), lambda b, t: (0, 0, 0)),
            pl.BlockSpec((B, NUM_SLOTS, D), lambda b, t: (0, 0, 0)),
            pl.BlockSpec((NUM_SLOTS, D), lambda b, t: (0, 0)),
            pl.BlockSpec((3 * D, D), lambda b, t: (0, 0)),
            pl.BlockSpec((3 * D, D), lambda b, t: (0, 0)),
            pl.BlockSpec((1, 3 * D), lambda b, t: (0, 0)),
            pl.BlockSpec((1, 3 * D), lambda b, t: (0, 0)),
            pl.BlockSpec((D, D), lambda b, t: (0, 0)),
            pl.BlockSpec((D, D), lambda b, t: (0, 0)),
            pl.BlockSpec((1, D), lambda b, t: (0, 0)),
            pl.BlockSpec((1, D), lambda b, t: (0, 0)),
        ],
        out_specs=pl.BlockSpec((1, TBLK, D), lambda b, t: (b, t, 0)),
        out_shape=jax.ShapeDtypeStruct((B, T, D), f32),
        scratch_shapes=[pltpu.VMEM((B, NUM_SLOTS, D), bf16)],
    )(alpha_t, alpha_t, slot_input, hp0, W_ih, W_hh,
      b_ih.reshape(1, 3 * D), b_hh.reshape(1, 3 * D), Wv, Wo,
      bv.reshape(1, D), bo.reshape(1, D))
    return out
